# Initial kernel scaffold; baseline (speedup 1.0000x reference)
import jax
import jax.numpy as jnp
from jax.experimental import pallas as pl


def kernel(x, edge_index, W1l, W1r, a1, b1, W2l, W2r, a2, b2):
    raise NotImplementedError("write your pallas kernel here")


# speedup vs baseline: 7.6518x; 7.6518x over previous
import functools

import jax
import jax.numpy as jnp
from jax import lax
from jax.experimental import pallas as pl
from jax.experimental.pallas import tpu as pltpu
from jax.experimental.pallas import tpu_sc as plsc

N = 10000
NP = 10240
E = 320000
ETOT = E + N
C = 128
NTILES = 32
CHUNKS = 81
T = CHUNKS * C
EPAD = NTILES * T
RPT = NP // 16

_SC_PARAMS = pltpu.CompilerParams(use_tc_tiling_on_sc=False)
_SC_MESH = dict(core_axis_name="c", subcore_axis_name="s")


def _lane_consts():
    lane = lax.iota(jnp.int32, 16)
    e0 = jnp.where(lane == 0, 1.0, 0.0)
    rot = [(lane + k) % 16 for k in (1, 2, 4, 8)]
    onehot = [jnp.where(lane == e, 1.0, 0.0) for e in range(16)]
    return lane, e0, rot, onehot


def _logit_w(xlv, xrv, av, eidx, rot, J):
    acc = None
    for j in range(J):
        v = xlv[eidx, pl.ds(j * 16, 16)] + xrv[eidx, pl.ds(j * 16, 16)]
        v = jnp.maximum(v, 0.2 * v)
        t = av[j] * v
        acc = t if acc is None else acc + t
    for r in rot:
        acc = acc + jnp.take(acc, r)
    return jnp.exp(acc)


def _sc_layer1_body(xl_hbm, xr_hbm, xlb_hbm, src_hbm, dst_hbm, a_hbm,
                    zacc_hbm,
                    accA_hbm, accB_hbm,
                    a_v, sidx, didx, xlv, xrv, outv, xlbv, outv2, wstore,
                    acc_s):
    c = lax.axis_index("c")
    s = lax.axis_index("s")
    wid = s * 2 + c
    pltpu.sync_copy(zacc_hbm.at[pl.ds(s * RPT, RPT)],
                    acc_s.at[pl.ds(s * RPT, RPT)])
    pltpu.sync_copy(a_hbm, a_v)
    zero16 = jnp.zeros((16,), jnp.float32)
    for e in range(C):
        outv2[e, pl.ds(64, 16)] = zero16
    plsc.subcore_barrier()
    av = [a_v[pl.ds(j * 16, 16)] for j in range(8)]
    lane, e0, rot, onehot = _lane_consts()

    def chunk1(i, carry):
        base = wid * T + i * C
        pltpu.sync_copy(src_hbm.at[pl.ds(base, C)], sidx)
        pltpu.sync_copy(dst_hbm.at[pl.ds(base, C)], didx)
        pltpu.sync_copy(xl_hbm.at[sidx], xlv)
        pltpu.sync_copy(xr_hbm.at[didx], xrv)

        def edge16(g, carry2):
            wpack = None
            for e in range(16):
                eidx = g * 16 + e
                w = _logit_w(xlv, xrv, av, eidx, rot, 8)
                for j in range(4):
                    outv[eidx, pl.ds(j * 16, 16)] = (
                        w * xlv[eidx, pl.ds(j * 16, 16)])
                outv[eidx, pl.ds(64, 16)] = w * e0
                t = w * onehot[e]
                wpack = t if wpack is None else wpack + t
            wstore[pl.ds(i * C + g * 16, 16)] = wpack
            return carry2

        lax.fori_loop(0, C // 16, edge16, 0)
        pltpu.sync_copy(outv, acc_s.at[didx], add=True)
        return carry

    lax.fori_loop(0, CHUNKS, chunk1, 0)
    plsc.subcore_barrier()
    pltpu.sync_copy(acc_s.at[pl.ds(s * RPT, RPT)],
                    accA_hbm.at[c, pl.ds(s * RPT, RPT)])
    pltpu.sync_copy(zacc_hbm.at[pl.ds(s * RPT, RPT)],
                    acc_s.at[pl.ds(s * RPT, RPT)])
    plsc.subcore_barrier()

    def chunk2(i, carry):
        base = wid * T + i * C
        pltpu.sync_copy(src_hbm.at[pl.ds(base, C)], sidx)
        pltpu.sync_copy(dst_hbm.at[pl.ds(base, C)], didx)
        pltpu.sync_copy(xlb_hbm.at[sidx], xlbv)

        def edge16(g, carry2):
            wpack = wstore[pl.ds(i * C + g * 16, 16)]
            for e in range(16):
                eidx = g * 16 + e
                w = jnp.full((16,), wpack[e])
                for j in range(4):
                    outv2[eidx, pl.ds(j * 16, 16)] = (
                        w * xlbv[eidx, pl.ds(j * 16, 16)])
            return carry2

        lax.fori_loop(0, C // 16, edge16, 0)
        pltpu.sync_copy(outv2, acc_s.at[didx], add=True)
        return carry

    lax.fori_loop(0, CHUNKS, chunk2, 0)
    plsc.subcore_barrier()
    pltpu.sync_copy(acc_s.at[pl.ds(s * RPT, RPT)],
                    accB_hbm.at[c, pl.ds(s * RPT, RPT)])


_sc_layer1 = pl.kernel(
    _sc_layer1_body,
    out_type=[
        jax.ShapeDtypeStruct((2, NP, 80), jnp.float32),
        jax.ShapeDtypeStruct((2, NP, 80), jnp.float32),
    ],
    mesh=plsc.VectorSubcoreMesh(**_SC_MESH),
    compiler_params=_SC_PARAMS,
    scratch_types=[
        pltpu.VMEM((128,), jnp.float32),
        pltpu.VMEM((C,), jnp.int32),
        pltpu.VMEM((C,), jnp.int32),
        pltpu.VMEM((C, 128), jnp.float32),
        pltpu.VMEM((C, 128), jnp.float32),
        pltpu.VMEM((C, 80), jnp.float32),
        pltpu.VMEM((C, 64), jnp.float32),
        pltpu.VMEM((C, 80), jnp.float32),
        pltpu.VMEM((T,), jnp.float32),
        pltpu.VMEM_SHARED((NP, 80), jnp.float32),
    ],
)


def _sc_layer2_body(xl_hbm, xr_hbm, src_hbm, dst_hbm, a_hbm, zacc_hbm,
                    acc_hbm,
                    a_v, sidx, didx, xlv, xrv, outv, acc_s):
    c = lax.axis_index("c")
    s = lax.axis_index("s")
    wid = s * 2 + c
    pltpu.sync_copy(zacc_hbm.at[pl.ds(s * RPT, RPT)],
                    acc_s.at[pl.ds(s * RPT, RPT)])
    pltpu.sync_copy(a_hbm, a_v)
    plsc.subcore_barrier()
    av = [a_v[pl.ds(0, 16)]]
    lane, e0, rot, onehot = _lane_consts()

    def chunk(i, carry):
        base = wid * T + i * C
        pltpu.sync_copy(src_hbm.at[pl.ds(base, C)], sidx)
        pltpu.sync_copy(dst_hbm.at[pl.ds(base, C)], didx)
        pltpu.sync_copy(xl_hbm.at[sidx], xlv)
        pltpu.sync_copy(xr_hbm.at[didx], xrv)

        def edge16(g, carry2):
            for e in range(16):
                eidx = g * 16 + e
                w = _logit_w(xlv, xrv, av, eidx, rot, 1)
                outv[eidx, pl.ds(0, 16)] = w * xlv[eidx, pl.ds(0, 16)]
                outv[eidx, pl.ds(16, 16)] = w * e0
            return carry2

        lax.fori_loop(0, C // 16, edge16, 0)
        pltpu.sync_copy(outv, acc_s.at[didx], add=True)
        return carry

    lax.fori_loop(0, CHUNKS, chunk, 0)
    plsc.subcore_barrier()
    pltpu.sync_copy(acc_s.at[pl.ds(s * RPT, RPT)],
                    acc_hbm.at[c, pl.ds(s * RPT, RPT)])


_sc_layer2 = pl.kernel(
    _sc_layer2_body,
    out_type=jax.ShapeDtypeStruct((2, NP, 32), jnp.float32),
    mesh=plsc.VectorSubcoreMesh(**_SC_MESH),
    compiler_params=_SC_PARAMS,
    scratch_types=[
        pltpu.VMEM((16,), jnp.float32),
        pltpu.VMEM((C,), jnp.int32),
        pltpu.VMEM((C,), jnp.int32),
        pltpu.VMEM((C, 16), jnp.float32),
        pltpu.VMEM((C, 16), jnp.float32),
        pltpu.VMEM((C, 32), jnp.float32),
        pltpu.VMEM_SHARED((NP, 32), jnp.float32),
    ],
)


def _mm1_kern(x_ref, wl_ref, wr_ref, xl_ref, xr_ref, xlb_ref):
    xv = x_ref[...]
    xl = jnp.dot(xv, wl_ref[...], preferred_element_type=jnp.float32)
    xl_ref[...] = xl
    xr_ref[...] = jnp.dot(xv, wr_ref[...], preferred_element_type=jnp.float32)
    xlb_ref[...] = xl[:, 64:]


def _h_kern(accA_ref, accB_ref, b1_ref, w2l_ref, w2r_ref, xl2_ref, xr2_ref):
    pA = accA_ref[0, :, :64] + accA_ref[1, :, :64]
    pB = accB_ref[0, :, :64] + accB_ref[1, :, :64]
    d = accA_ref[0, :, 64:65] + accA_ref[1, :, 64:65] + 1e-16
    h = jnp.concatenate([pA, pB], axis=1) / d + b1_ref[...]
    h = jnp.where(h > 0, h, jnp.exp(h) - 1.0)
    xl2_ref[...] = jnp.dot(h, w2l_ref[...], preferred_element_type=jnp.float32)
    xr2_ref[...] = jnp.dot(h, w2r_ref[...], preferred_element_type=jnp.float32)


def _fin_kern(acc_ref, b2_ref, h_ref, y_ref):
    p = acc_ref[0, :, :16] + acc_ref[1, :, :16]
    d = acc_ref[0, :, 16:17] + acc_ref[1, :, 16:17] + 1e-16
    h = p / d + b2_ref[...]
    m = jnp.max(h, axis=1, keepdims=True)
    ex = jnp.exp(h - m)
    lse = jnp.log(jnp.sum(ex, axis=1, keepdims=True))
    h_ref[...] = h
    y_ref[...] = h - m - lse


def kernel(x, edge_index, W1l, W1r, a1, b1, W2l, W2r, a2, b2):
    f32 = jnp.float32
    loops = jnp.arange(N, dtype=jnp.int32)
    src = jnp.concatenate([edge_index[0].astype(jnp.int32), loops,
                           jnp.zeros((EPAD - ETOT,), jnp.int32)])
    dst = jnp.concatenate([edge_index[1].astype(jnp.int32), loops,
                           jnp.full((EPAD - ETOT,), N, jnp.int32)])
    x_p = jnp.pad(x, ((0, NP - N), (0, 0)))
    zacc1 = jnp.zeros((NP, 80), f32)
    zacc2 = jnp.zeros((NP, 32), f32)

    xl1, xr1, xlb1 = pl.pallas_call(
        _mm1_kern,
        out_shape=[jax.ShapeDtypeStruct((NP, 128), f32)] * 2
        + [jax.ShapeDtypeStruct((NP, 64), f32)],
    )(x_p, W1l, W1r)

    accA, accB = _sc_layer1(xl1, xr1, xlb1, src, dst, a1, zacc1)

    xl2, xr2 = pl.pallas_call(
        _h_kern,
        out_shape=[jax.ShapeDtypeStruct((NP, 16), f32)] * 2,
    )(accA, accB, b1.reshape(1, 128), W2l, W2r)

    acc2 = _sc_layer2(xl2, xr2, src, dst, a2, zacc2)

    h2, y = pl.pallas_call(
        _fin_kern,
        out_shape=[jax.ShapeDtypeStruct((NP, 16), f32)] * 2,
    )(acc2, b2.reshape(1, 16))

    return (h2[:N], y[:N])


# speedup vs baseline: 9.9062x; 1.2946x over previous
import jax
import jax.numpy as jnp
from jax import lax
from jax.experimental import pallas as pl
from jax.experimental.pallas import tpu as pltpu
from jax.experimental.pallas import tpu_sc as plsc

N = 10000
NP = 10240
E = 320000
ETOT = E + N
C = 64
NTILES = 32
CHUNKS = 162
T = CHUNKS * C
EPAD = NTILES * T
IPAD = EPAD + C
RPT = NP // 16

_SC_PARAMS = pltpu.CompilerParams(use_tc_tiling_on_sc=False)
_SC_MESH = dict(core_axis_name="c", subcore_axis_name="s")


def _lane_consts():
    lane = lax.iota(jnp.int32, 16)
    e0 = jnp.where(lane == 0, 1.0, 0.0)
    rot = [(lane + k) % 16 for k in (1, 2, 4, 8)]
    onehot = [jnp.where(lane == e, 1.0, 0.0) for e in range(16)]
    return lane, e0, rot, onehot


def _prefetch(xl_hbm, xr_hbm, src_hbm, dst_hbm, base, sidx, didx, xlv, xrv,
              sem):
    pltpu.sync_copy(src_hbm.at[pl.ds(base, C)], sidx)
    pltpu.sync_copy(dst_hbm.at[pl.ds(base, C)], didx)
    pltpu.async_copy(xl_hbm.at[sidx], xlv, sem)
    pltpu.async_copy(xr_hbm.at[didx], xrv, sem)


def _drain(xl_hbm, xr_hbm, sidx, didx, xlv, xrv, sem):
    pltpu.make_async_copy(xl_hbm.at[sidx], xlv, sem).wait()
    pltpu.make_async_copy(xr_hbm.at[didx], xrv, sem).wait()


def _sc_layer1_body(xl_hbm, xr_hbm, xlb_hbm, src_hbm, dst_hbm, a_hbm,
                    zacc_hbm,
                    accA_hbm, accB_hbm,
                    a_v, sidx0, didx0, xlv0, xrv0, sidx1, didx1, xlv1, xrv1,
                    outv, xlbv0, xlbv1, outv2, wstore, gsem0, gsem1, acc_s):
    c = lax.axis_index("c")
    s = lax.axis_index("s")
    wid = s * 2 + c
    pltpu.sync_copy(zacc_hbm.at[pl.ds(s * RPT, RPT)],
                    acc_s.at[pl.ds(s * RPT, RPT)])
    pltpu.sync_copy(a_hbm, a_v)
    zero16 = jnp.zeros((16,), jnp.float32)
    for e in range(C):
        outv2[e, pl.ds(64, 16)] = zero16
    plsc.subcore_barrier()
    av = [a_v[pl.ds(j * 16, 16)] for j in range(8)]
    lane, e0, rot, onehot = _lane_consts()
    tbase = wid * T

    def compute1(i, sidx, didx, xlv, xrv):
        def edge16(g, carry2):
            wpack = None
            for e in range(16):
                eidx = g * 16 + e
                xa = [xlv[eidx, pl.ds(j * 16, 16)] for j in range(4)]
                acc = None
                for j in range(8):
                    xj = xa[j] if j < 4 else xlv[eidx, pl.ds(j * 16, 16)]
                    v = xj + xrv[eidx, pl.ds(j * 16, 16)]
                    v = jnp.maximum(v, 0.2 * v)
                    t = av[j] * v
                    acc = t if acc is None else acc + t
                for r in rot:
                    acc = acc + jnp.take(acc, r)
                w = jnp.exp(acc)
                for j in range(4):
                    outv[eidx, pl.ds(j * 16, 16)] = w * xa[j]
                outv[eidx, pl.ds(64, 16)] = w * e0
                t = w * onehot[e]
                wpack = t if wpack is None else wpack + t
            wstore[pl.ds(i * C + g * 16, 16)] = wpack
            return carry2

        lax.fori_loop(0, C // 16, edge16, 0)
        pltpu.sync_copy(outv, acc_s.at[didx], add=True)

    _prefetch(xl_hbm, xr_hbm, src_hbm, dst_hbm, tbase, sidx0, didx0,
              xlv0, xrv0, gsem0)

    def pair1(k, carry):
        a = 2 * k
        _prefetch(xl_hbm, xr_hbm, src_hbm, dst_hbm, tbase + (a + 1) * C,
                  sidx1, didx1, xlv1, xrv1, gsem1)
        _drain(xl_hbm, xr_hbm, sidx0, didx0, xlv0, xrv0, gsem0)
        compute1(a, sidx0, didx0, xlv0, xrv0)
        _prefetch(xl_hbm, xr_hbm, src_hbm, dst_hbm, tbase + (a + 2) * C,
                  sidx0, didx0, xlv0, xrv0, gsem0)
        _drain(xl_hbm, xr_hbm, sidx1, didx1, xlv1, xrv1, gsem1)
        compute1(a + 1, sidx1, didx1, xlv1, xrv1)
        return carry

    lax.fori_loop(0, CHUNKS // 2, pair1, 0)
    _drain(xl_hbm, xr_hbm, sidx0, didx0, xlv0, xrv0, gsem0)
    plsc.subcore_barrier()
    pltpu.sync_copy(acc_s.at[pl.ds(s * RPT, RPT)],
                    accA_hbm.at[c, pl.ds(s * RPT, RPT)])
    pltpu.sync_copy(zacc_hbm.at[pl.ds(s * RPT, RPT)],
                    acc_s.at[pl.ds(s * RPT, RPT)])
    plsc.subcore_barrier()

    def pre2(base, sidx, didx, xlbv, sem):
        pltpu.sync_copy(src_hbm.at[pl.ds(base, C)], sidx)
        pltpu.sync_copy(dst_hbm.at[pl.ds(base, C)], didx)
        pltpu.async_copy(xlb_hbm.at[sidx], xlbv, sem)

    def drain2(sidx, xlbv, sem):
        pltpu.make_async_copy(xlb_hbm.at[sidx], xlbv, sem).wait()

    def compute2(i, didx, xlbv):
        def edge16(g, carry2):
            wpack = wstore[pl.ds(i * C + g * 16, 16)]
            for e in range(16):
                eidx = g * 16 + e
                w = jnp.full((16,), wpack[e])
                for j in range(4):
                    outv2[eidx, pl.ds(j * 16, 16)] = (
                        w * xlbv[eidx, pl.ds(j * 16, 16)])
            return carry2

        lax.fori_loop(0, C // 16, edge16, 0)
        pltpu.sync_copy(outv2, acc_s.at[didx], add=True)

    pre2(tbase, sidx0, didx0, xlbv0, gsem0)

    def pair2(k, carry):
        a = 2 * k
        pre2(tbase + (a + 1) * C, sidx1, didx1, xlbv1, gsem1)
        drain2(sidx0, xlbv0, gsem0)
        compute2(a, didx0, xlbv0)
        pre2(tbase + (a + 2) * C, sidx0, didx0, xlbv0, gsem0)
        drain2(sidx1, xlbv1, gsem1)
        compute2(a + 1, didx1, xlbv1)
        return carry

    lax.fori_loop(0, CHUNKS // 2, pair2, 0)
    drain2(sidx0, xlbv0, gsem0)
    plsc.subcore_barrier()
    pltpu.sync_copy(acc_s.at[pl.ds(s * RPT, RPT)],
                    accB_hbm.at[c, pl.ds(s * RPT, RPT)])


_sc_layer1 = pl.kernel(
    _sc_layer1_body,
    out_type=[
        jax.ShapeDtypeStruct((2, NP, 80), jnp.float32),
        jax.ShapeDtypeStruct((2, NP, 80), jnp.float32),
    ],
    mesh=plsc.VectorSubcoreMesh(**_SC_MESH),
    compiler_params=_SC_PARAMS,
    scratch_types=[
        pltpu.VMEM((128,), jnp.float32),
        pltpu.VMEM((C,), jnp.int32),
        pltpu.VMEM((C,), jnp.int32),
        pltpu.VMEM((C, 128), jnp.float32),
        pltpu.VMEM((C, 128), jnp.float32),
        pltpu.VMEM((C,), jnp.int32),
        pltpu.VMEM((C,), jnp.int32),
        pltpu.VMEM((C, 128), jnp.float32),
        pltpu.VMEM((C, 128), jnp.float32),
        pltpu.VMEM((C, 80), jnp.float32),
        pltpu.VMEM((C, 64), jnp.float32),
        pltpu.VMEM((C, 64), jnp.float32),
        pltpu.VMEM((C, 80), jnp.float32),
        pltpu.VMEM((T,), jnp.float32),
        pltpu.SemaphoreType.DMA,
        pltpu.SemaphoreType.DMA,
        pltpu.VMEM_SHARED((NP, 80), jnp.float32),
    ],
)


def _sc_layer2_body(xl_hbm, xr_hbm, src_hbm, dst_hbm, a_hbm, zacc_hbm,
                    acc_hbm,
                    a_v, sidx0, didx0, xlv0, xrv0, sidx1, didx1, xlv1, xrv1,
                    outv, gsem0, gsem1, acc_s):
    c = lax.axis_index("c")
    s = lax.axis_index("s")
    wid = s * 2 + c
    pltpu.sync_copy(zacc_hbm.at[pl.ds(s * RPT, RPT)],
                    acc_s.at[pl.ds(s * RPT, RPT)])
    pltpu.sync_copy(a_hbm, a_v)
    plsc.subcore_barrier()
    av = a_v[pl.ds(0, 16)]
    lane, e0, rot, onehot = _lane_consts()
    tbase = wid * T

    def compute(i, didx, xlv, xrv):
        def edge16(g, carry2):
            for e in range(16):
                eidx = g * 16 + e
                x0 = xlv[eidx, pl.ds(0, 16)]
                v = x0 + xrv[eidx, pl.ds(0, 16)]
                v = jnp.maximum(v, 0.2 * v)
                acc = av * v
                for r in rot:
                    acc = acc + jnp.take(acc, r)
                w = jnp.exp(acc)
                outv[eidx, pl.ds(0, 16)] = w * x0
                outv[eidx, pl.ds(16, 16)] = w * e0
            return carry2

        lax.fori_loop(0, C // 16, edge16, 0)
        pltpu.sync_copy(outv, acc_s.at[didx], add=True)

    _prefetch(xl_hbm, xr_hbm, src_hbm, dst_hbm, tbase, sidx0, didx0,
              xlv0, xrv0, gsem0)

    def pair(k, carry):
        a = 2 * k
        _prefetch(xl_hbm, xr_hbm, src_hbm, dst_hbm, tbase + (a + 1) * C,
                  sidx1, didx1, xlv1, xrv1, gsem1)
        _drain(xl_hbm, xr_hbm, sidx0, didx0, xlv0, xrv0, gsem0)
        compute(a, didx0, xlv0, xrv0)
        _prefetch(xl_hbm, xr_hbm, src_hbm, dst_hbm, tbase + (a + 2) * C,
                  sidx0, didx0, xlv0, xrv0, gsem0)
        _drain(xl_hbm, xr_hbm, sidx1, didx1, xlv1, xrv1, gsem1)
        compute(a + 1, didx1, xlv1, xrv1)
        return carry

    lax.fori_loop(0, CHUNKS // 2, pair, 0)
    _drain(xl_hbm, xr_hbm, sidx0, didx0, xlv0, xrv0, gsem0)
    plsc.subcore_barrier()
    pltpu.sync_copy(acc_s.at[pl.ds(s * RPT, RPT)],
                    acc_hbm.at[c, pl.ds(s * RPT, RPT)])


_sc_layer2 = pl.kernel(
    _sc_layer2_body,
    out_type=jax.ShapeDtypeStruct((2, NP, 32), jnp.float32),
    mesh=plsc.VectorSubcoreMesh(**_SC_MESH),
    compiler_params=_SC_PARAMS,
    scratch_types=[
        pltpu.VMEM((16,), jnp.float32),
        pltpu.VMEM((C,), jnp.int32),
        pltpu.VMEM((C,), jnp.int32),
        pltpu.VMEM((C, 16), jnp.float32),
        pltpu.VMEM((C, 16), jnp.float32),
        pltpu.VMEM((C,), jnp.int32),
        pltpu.VMEM((C,), jnp.int32),
        pltpu.VMEM((C, 16), jnp.float32),
        pltpu.VMEM((C, 16), jnp.float32),
        pltpu.VMEM((C, 32), jnp.float32),
        pltpu.SemaphoreType.DMA,
        pltpu.SemaphoreType.DMA,
        pltpu.VMEM_SHARED((NP, 32), jnp.float32),
    ],
)


def _mm1_kern(x_ref, wl_ref, wr_ref, xl_ref, xr_ref, xlb_ref):
    xv = x_ref[...]
    xl = jnp.dot(xv, wl_ref[...], preferred_element_type=jnp.float32)
    xl_ref[...] = xl
    xr_ref[...] = jnp.dot(xv, wr_ref[...], preferred_element_type=jnp.float32)
    xlb_ref[...] = xl[:, 64:]


def _h_kern(accA_ref, accB_ref, b1_ref, w2l_ref, w2r_ref, xl2_ref, xr2_ref):
    pA = accA_ref[0, :, :64] + accA_ref[1, :, :64]
    pB = accB_ref[0, :, :64] + accB_ref[1, :, :64]
    d = accA_ref[0, :, 64:65] + accA_ref[1, :, 64:65] + 1e-16
    h = jnp.concatenate([pA, pB], axis=1) / d + b1_ref[...]
    h = jnp.where(h > 0, h, jnp.exp(h) - 1.0)
    xl2_ref[...] = jnp.dot(h, w2l_ref[...], preferred_element_type=jnp.float32)
    xr2_ref[...] = jnp.dot(h, w2r_ref[...], preferred_element_type=jnp.float32)


def _fin_kern(acc_ref, b2_ref, h_ref, y_ref):
    p = acc_ref[0, :, :16] + acc_ref[1, :, :16]
    d = acc_ref[0, :, 16:17] + acc_ref[1, :, 16:17] + 1e-16
    h = p / d + b2_ref[...]
    m = jnp.max(h, axis=1, keepdims=True)
    ex = jnp.exp(h - m)
    lse = jnp.log(jnp.sum(ex, axis=1, keepdims=True))
    h_ref[...] = h
    y_ref[...] = h - m - lse


def kernel(x, edge_index, W1l, W1r, a1, b1, W2l, W2r, a2, b2):
    f32 = jnp.float32
    loops = jnp.arange(N, dtype=jnp.int32)
    src = jnp.concatenate([edge_index[0].astype(jnp.int32), loops,
                           jnp.zeros((IPAD - ETOT,), jnp.int32)])
    dst = jnp.concatenate([edge_index[1].astype(jnp.int32), loops,
                           jnp.full((EPAD - ETOT,), N, jnp.int32),
                           jnp.full((IPAD - EPAD,), N, jnp.int32)])
    x_p = jnp.pad(x, ((0, NP - N), (0, 0)))
    zacc1 = jnp.zeros((NP, 80), f32)
    zacc2 = jnp.zeros((NP, 32), f32)

    xl1, xr1, xlb1 = pl.pallas_call(
        _mm1_kern,
        out_shape=[jax.ShapeDtypeStruct((NP, 128), f32)] * 2
        + [jax.ShapeDtypeStruct((NP, 64), f32)],
    )(x_p, W1l, W1r)

    accA, accB = _sc_layer1(xl1, xr1, xlb1, src, dst, a1, zacc1)

    xl2, xr2 = pl.pallas_call(
        _h_kern,
        out_shape=[jax.ShapeDtypeStruct((NP, 16), f32)] * 2,
    )(accA, accB, b1.reshape(1, 128), W2l, W2r)

    acc2 = _sc_layer2(xl2, xr2, src, dst, a2, zacc2)

    h2, y = pl.pallas_call(
        _fin_kern,
        out_shape=[jax.ShapeDtypeStruct((NP, 16), f32)] * 2,
    )(acc2, b2.reshape(1, 16))

    return (h2[:N], y[:N])


# speedup vs baseline: 13.8845x; 1.4016x over previous
import jax
import jax.numpy as jnp
from jax import lax
from jax.experimental import pallas as pl
from jax.experimental.pallas import tpu as pltpu
from jax.experimental.pallas import tpu_sc as plsc

N = 10000
NP = 10240
E = 320000
ETOT = E + N
C = 64
NTILES = 32
CHUNKS = 162
T = CHUNKS * C
EPAD = NTILES * T
IPAD = EPAD + C
NROW = IPAD // C
RPT = NP // 16

_SC_PARAMS = pltpu.CompilerParams(use_tc_tiling_on_sc=False)
_SC_MESH = dict(core_axis_name="c", subcore_axis_name="s")


def _lane_consts():
    lane = lax.iota(jnp.int32, 16)
    e0 = jnp.where(lane == 0, 1.0, 0.0)
    rot = [(lane + k) % 16 for k in (1, 2, 4, 8)]
    onehot = [jnp.where(lane == e, 1.0, 0.0) for e in range(16)]
    return lane, e0, rot, onehot


def _sc_layer1_body(xl_hbm, xr_hbm, src_hbm, dst_hbm, a_hbm, zacc_hbm,
                    accA_hbm, accB_hbm,
                    a_v, srcall, dstall, xlv0, xrv0, xlv1, xrv1,
                    outv0, outv1, wstore, gsem0, gsem1, ssem0, ssem1, acc_s):
    c = lax.axis_index("c")
    s = lax.axis_index("s")
    wid = s * 2 + c
    rowbase = wid * CHUNKS
    pltpu.sync_copy(zacc_hbm.at[pl.ds(s * RPT, RPT)],
                    acc_s.at[pl.ds(s * RPT, RPT)])
    pltpu.sync_copy(a_hbm, a_v)
    pltpu.sync_copy(src_hbm.at[pl.ds(rowbase, CHUNKS + 1)], srcall)
    pltpu.sync_copy(dst_hbm.at[pl.ds(rowbase, CHUNKS + 1)], dstall)
    plsc.subcore_barrier()
    av = [a_v[pl.ds(j * 16, 16)] for j in range(8)]
    lane, e0, rot, onehot = _lane_consts()

    def gath(i, xlv, xrv, sem):
        pltpu.async_copy(xl_hbm.at[srcall.at[i]], xlv, sem)
        pltpu.async_copy(xr_hbm.at[dstall.at[i]], xrv, sem)

    def gdrain(i, xlv, xrv, sem):
        pltpu.make_async_copy(xl_hbm.at[srcall.at[i]], xlv, sem).wait()
        pltpu.make_async_copy(xr_hbm.at[dstall.at[i]], xrv, sem).wait()

    def sdrain(outv, sem):
        pltpu.make_async_copy(outv, acc_s.at[dstall.at[0]], sem).wait()

    def compute1(i, xlv, xrv, outv):
        def edge16(g, carry2):
            wpack = None
            for e in range(16):
                eidx = g * 16 + e
                xa = [xlv[eidx, pl.ds(j * 16, 16)] for j in range(4)]
                acc = None
                for j in range(8):
                    xj = xa[j] if j < 4 else xlv[eidx, pl.ds(j * 16, 16)]
                    v = xj + xrv[eidx, pl.ds(j * 16, 16)]
                    v = jnp.maximum(v, 0.2 * v)
                    t = av[j] * v
                    acc = t if acc is None else acc + t
                for r in rot:
                    acc = acc + jnp.take(acc, r)
                w = jnp.exp(acc)
                for j in range(4):
                    outv[eidx, pl.ds(j * 16, 16)] = w * xa[j]
                outv[eidx, pl.ds(64, 16)] = w * e0
                t = w * onehot[e]
                wpack = t if wpack is None else wpack + t
            wstore[pl.ds(i * C + g * 16, 16)] = wpack
            return carry2

        lax.fori_loop(0, C // 16, edge16, 0)

    def scat(i, outv, sem):
        pltpu.async_copy(outv, acc_s.at[dstall.at[i]], sem, add=True)

    gath(0, xlv0, xrv0, gsem0)

    def pair1(k, carry):
        a = 2 * k

        @pl.when(k > 0)
        def _():
            sdrain(outv0, ssem0)
            sdrain(outv1, ssem1)

        gath(a + 1, xlv1, xrv1, gsem1)
        gdrain(a, xlv0, xrv0, gsem0)
        compute1(a, xlv0, xrv0, outv0)
        scat(a, outv0, ssem0)
        gath(a + 2, xlv0, xrv0, gsem0)
        gdrain(a + 1, xlv1, xrv1, gsem1)
        compute1(a + 1, xlv1, xrv1, outv1)
        scat(a + 1, outv1, ssem1)
        return carry

    lax.fori_loop(0, CHUNKS // 2, pair1, 0)
    gdrain(0, xlv0, xrv0, gsem0)
    sdrain(outv0, ssem0)
    sdrain(outv1, ssem1)
    plsc.subcore_barrier()
    pltpu.sync_copy(acc_s.at[pl.ds(s * RPT, RPT)],
                    accA_hbm.at[c, pl.ds(s * RPT, RPT)])
    pltpu.sync_copy(zacc_hbm.at[pl.ds(s * RPT, RPT)],
                    acc_s.at[pl.ds(s * RPT, RPT)])
    zero16 = jnp.zeros((16,), jnp.float32)
    for e in range(C):
        outv0[e, pl.ds(64, 16)] = zero16
        outv1[e, pl.ds(64, 16)] = zero16
    plsc.subcore_barrier()

    def gath2(i, xlv, sem):
        pltpu.async_copy(xl_hbm.at[srcall.at[i]], xlv, sem)

    def gdrain2(i, xlv, sem):
        pltpu.make_async_copy(xl_hbm.at[srcall.at[i]], xlv, sem).wait()

    def compute2(i, xlv, outv):
        def edge16(g, carry2):
            wpack = wstore[pl.ds(i * C + g * 16, 16)]
            for e in range(16):
                eidx = g * 16 + e
                w = jnp.full((16,), wpack[e])
                for j in range(4):
                    outv[eidx, pl.ds(j * 16, 16)] = (
                        w * xlv[eidx, pl.ds((4 + j) * 16, 16)])
            return carry2

        lax.fori_loop(0, C // 16, edge16, 0)

    gath2(0, xlv0, gsem0)

    def pair2(k, carry):
        a = 2 * k

        @pl.when(k > 0)
        def _():
            sdrain(outv0, ssem0)
            sdrain(outv1, ssem1)

        gath2(a + 1, xlv1, gsem1)
        gdrain2(a, xlv0, gsem0)
        compute2(a, xlv0, outv0)
        scat(a, outv0, ssem0)
        gath2(a + 2, xlv0, gsem0)
        gdrain2(a + 1, xlv1, gsem1)
        compute2(a + 1, xlv1, outv1)
        scat(a + 1, outv1, ssem1)
        return carry

    lax.fori_loop(0, CHUNKS // 2, pair2, 0)
    gdrain2(0, xlv0, gsem0)
    sdrain(outv0, ssem0)
    sdrain(outv1, ssem1)
    plsc.subcore_barrier()
    pltpu.sync_copy(acc_s.at[pl.ds(s * RPT, RPT)],
                    accB_hbm.at[c, pl.ds(s * RPT, RPT)])


_sc_layer1 = pl.kernel(
    _sc_layer1_body,
    out_type=[
        jax.ShapeDtypeStruct((2, NP, 80), jnp.float32),
        jax.ShapeDtypeStruct((2, NP, 80), jnp.float32),
    ],
    mesh=plsc.VectorSubcoreMesh(**_SC_MESH),
    compiler_params=_SC_PARAMS,
    scratch_types=[
        pltpu.VMEM((128,), jnp.float32),
        pltpu.VMEM((CHUNKS + 1, C), jnp.int32),
        pltpu.VMEM((CHUNKS + 1, C), jnp.int32),
        pltpu.VMEM((C, 128), jnp.float32),
        pltpu.VMEM((C, 128), jnp.float32),
        pltpu.VMEM((C, 128), jnp.float32),
        pltpu.VMEM((C, 128), jnp.float32),
        pltpu.VMEM((C, 80), jnp.float32),
        pltpu.VMEM((C, 80), jnp.float32),
        pltpu.VMEM((T,), jnp.float32),
        pltpu.SemaphoreType.DMA,
        pltpu.SemaphoreType.DMA,
        pltpu.SemaphoreType.DMA,
        pltpu.SemaphoreType.DMA,
        pltpu.VMEM_SHARED((NP, 80), jnp.float32),
    ],
)


def _sc_layer2_body(xl_hbm, xr_hbm, src_hbm, dst_hbm, a_hbm, zacc_hbm,
                    acc_hbm,
                    a_v, srcall, dstall, xlv0, xrv0, xlv1, xrv1,
                    outv0, outv1, gsem0, gsem1, ssem0, ssem1, acc_s):
    c = lax.axis_index("c")
    s = lax.axis_index("s")
    wid = s * 2 + c
    rowbase = wid * CHUNKS
    pltpu.sync_copy(zacc_hbm.at[pl.ds(s * RPT, RPT)],
                    acc_s.at[pl.ds(s * RPT, RPT)])
    pltpu.sync_copy(a_hbm, a_v)
    pltpu.sync_copy(src_hbm.at[pl.ds(rowbase, CHUNKS + 1)], srcall)
    pltpu.sync_copy(dst_hbm.at[pl.ds(rowbase, CHUNKS + 1)], dstall)
    plsc.subcore_barrier()
    av = a_v[pl.ds(0, 16)]
    lane, e0, rot, onehot = _lane_consts()

    def gath(i, xlv, xrv, sem):
        pltpu.async_copy(xl_hbm.at[srcall.at[i]], xlv, sem)
        pltpu.async_copy(xr_hbm.at[dstall.at[i]], xrv, sem)

    def gdrain(i, xlv, xrv, sem):
        pltpu.make_async_copy(xl_hbm.at[srcall.at[i]], xlv, sem).wait()
        pltpu.make_async_copy(xr_hbm.at[dstall.at[i]], xrv, sem).wait()

    def sdrain(outv, sem):
        pltpu.make_async_copy(outv, acc_s.at[dstall.at[0]], sem).wait()

    def compute(i, xlv, xrv, outv):
        def edge16(g, carry2):
            for e in range(16):
                eidx = g * 16 + e
                x0 = xlv[eidx, pl.ds(0, 16)]
                v = x0 + xrv[eidx, pl.ds(0, 16)]
                v = jnp.maximum(v, 0.2 * v)
                acc = av * v
                for r in rot:
                    acc = acc + jnp.take(acc, r)
                w = jnp.exp(acc)
                outv[eidx, pl.ds(0, 16)] = w * x0
                outv[eidx, pl.ds(16, 16)] = w * e0
            return carry2

        lax.fori_loop(0, C // 16, edge16, 0)

    def scat(i, outv, sem):
        pltpu.async_copy(outv, acc_s.at[dstall.at[i]], sem, add=True)

    gath(0, xlv0, xrv0, gsem0)

    def pair(k, carry):
        a = 2 * k

        @pl.when(k > 0)
        def _():
            sdrain(outv0, ssem0)
            sdrain(outv1, ssem1)

        gath(a + 1, xlv1, xrv1, gsem1)
        gdrain(a, xlv0, xrv0, gsem0)
        compute(a, xlv0, xrv0, outv0)
        scat(a, outv0, ssem0)
        gath(a + 2, xlv0, xrv0, gsem0)
        gdrain(a + 1, xlv1, xrv1, gsem1)
        compute(a + 1, xlv1, xrv1, outv1)
        scat(a + 1, outv1, ssem1)
        return carry

    lax.fori_loop(0, CHUNKS // 2, pair, 0)
    gdrain(0, xlv0, xrv0, gsem0)
    sdrain(outv0, ssem0)
    sdrain(outv1, ssem1)
    plsc.subcore_barrier()
    pltpu.sync_copy(acc_s.at[pl.ds(s * RPT, RPT)],
                    acc_hbm.at[c, pl.ds(s * RPT, RPT)])


_sc_layer2 = pl.kernel(
    _sc_layer2_body,
    out_type=jax.ShapeDtypeStruct((2, NP, 32), jnp.float32),
    mesh=plsc.VectorSubcoreMesh(**_SC_MESH),
    compiler_params=_SC_PARAMS,
    scratch_types=[
        pltpu.VMEM((16,), jnp.float32),
        pltpu.VMEM((CHUNKS + 1, C), jnp.int32),
        pltpu.VMEM((CHUNKS + 1, C), jnp.int32),
        pltpu.VMEM((C, 16), jnp.float32),
        pltpu.VMEM((C, 16), jnp.float32),
        pltpu.VMEM((C, 16), jnp.float32),
        pltpu.VMEM((C, 16), jnp.float32),
        pltpu.VMEM((C, 32), jnp.float32),
        pltpu.VMEM((C, 32), jnp.float32),
        pltpu.SemaphoreType.DMA,
        pltpu.SemaphoreType.DMA,
        pltpu.SemaphoreType.DMA,
        pltpu.SemaphoreType.DMA,
        pltpu.VMEM_SHARED((NP, 32), jnp.float32),
    ],
)


def _mm1_kern(x_ref, wl_ref, wr_ref, xl_ref, xr_ref):
    xv = x_ref[...]
    xl_ref[...] = jnp.dot(xv, wl_ref[...], preferred_element_type=jnp.float32)
    xr_ref[...] = jnp.dot(xv, wr_ref[...], preferred_element_type=jnp.float32)


def _h_kern(accA_ref, accB_ref, b1_ref, w2l_ref, w2r_ref, xl2_ref, xr2_ref):
    pA = accA_ref[0, :, :64] + accA_ref[1, :, :64]
    pB = accB_ref[0, :, :64] + accB_ref[1, :, :64]
    d = accA_ref[0, :, 64:65] + accA_ref[1, :, 64:65] + 1e-16
    h = jnp.concatenate([pA, pB], axis=1) / d + b1_ref[...]
    h = jnp.where(h > 0, h, jnp.exp(h) - 1.0)
    xl2_ref[...] = jnp.dot(h, w2l_ref[...], preferred_element_type=jnp.float32)
    xr2_ref[...] = jnp.dot(h, w2r_ref[...], preferred_element_type=jnp.float32)


def _fin_kern(acc_ref, b2_ref, h_ref, y_ref):
    p = acc_ref[0, :, :16] + acc_ref[1, :, :16]
    d = acc_ref[0, :, 16:17] + acc_ref[1, :, 16:17] + 1e-16
    h = p / d + b2_ref[...]
    m = jnp.max(h, axis=1, keepdims=True)
    ex = jnp.exp(h - m)
    lse = jnp.log(jnp.sum(ex, axis=1, keepdims=True))
    h_ref[...] = h
    y_ref[...] = h - m - lse


def kernel(x, edge_index, W1l, W1r, a1, b1, W2l, W2r, a2, b2):
    f32 = jnp.float32
    loops = jnp.arange(N, dtype=jnp.int32)
    src = jnp.concatenate([edge_index[0].astype(jnp.int32), loops,
                           jnp.zeros((IPAD - ETOT,), jnp.int32)])
    dst = jnp.concatenate([edge_index[1].astype(jnp.int32), loops,
                           jnp.full((IPAD - ETOT,), N, jnp.int32)])
    src2d = src.reshape(NROW, C)
    dst2d = dst.reshape(NROW, C)
    x_p = jnp.pad(x, ((0, NP - N), (0, 0)))
    zacc1 = jnp.zeros((NP, 80), f32)
    zacc2 = jnp.zeros((NP, 32), f32)

    xl1, xr1 = pl.pallas_call(
        _mm1_kern,
        out_shape=[jax.ShapeDtypeStruct((NP, 128), f32)] * 2,
    )(x_p, W1l, W1r)

    accA, accB = _sc_layer1(xl1, xr1, src2d, dst2d, a1, zacc1)

    xl2, xr2 = pl.pallas_call(
        _h_kern,
        out_shape=[jax.ShapeDtypeStruct((NP, 16), f32)] * 2,
    )(accA, accB, b1.reshape(1, 128), W2l, W2r)

    acc2 = _sc_layer2(xl2, xr2, src2d, dst2d, a2, zacc2)

    h2, y = pl.pallas_call(
        _fin_kern,
        out_shape=[jax.ShapeDtypeStruct((NP, 16), f32)] * 2,
    )(acc2, b2.reshape(1, 16))

    return (h2[:N], y[:N])
